# Initial kernel scaffold; baseline (speedup 1.0000x reference)
import functools

import jax
import jax.numpy as jnp
from jax import lax
from jax.experimental import pallas as pl
from jax.experimental.pallas import tpu as pltpu
from jax.experimental.pallas import tpu_sc as plsc

NC = 2
NS = 16
NW = NC * NS
CH = 128
DW = 16

_MESH = plsc.VectorSubcoreMesh(core_axis_name="c", subcore_axis_name="s")


def _ceil_to(v, m):
    return -(-v // m) * m


@functools.lru_cache(maxsize=None)
def _make_deg_kernel(n_pad, k):
    rps = n_pad // NS
    zb = rps // CH

    @functools.partial(
        pl.kernel,
        mesh=_MESH,
        out_type=jax.ShapeDtypeStruct((NC, n_pad, DW), jnp.float32),
        scratch_types=[
            pltpu.VMEM((k, CH), jnp.int32),
            pltpu.VMEM((CH, DW), jnp.float32),
            pltpu.VMEM((CH, DW), jnp.float32),
            pltpu.VMEM_SHARED((n_pad, DW), jnp.float32),
        ],
    )
    def deg_kernel(dst_hbm, ones_hbm, zeros_hbm, out_hbm, dst_v, ones_v, zero_v, shared):
        c = lax.axis_index("c")
        s = lax.axis_index("s")
        wid = c * NS + s
        pltpu.sync_copy(dst_hbm.at[wid], dst_v)
        pltpu.sync_copy(ones_hbm, ones_v)
        pltpu.sync_copy(zeros_hbm, zero_v)

        @pl.loop(0, zb)
        def _(z):
            pltpu.sync_copy(zero_v, shared.at[pl.ds(s * rps + z * CH, CH)])

        plsc.subcore_barrier()

        @pl.loop(0, k)
        def _(j):
            pltpu.sync_copy(ones_v, shared.at[dst_v.at[j]], add=True)

        plsc.subcore_barrier()

        @pl.loop(0, zb)
        def _(z):
            r0 = s * rps + z * CH
            pltpu.sync_copy(shared.at[pl.ds(r0, CH)], zero_v)
            pltpu.sync_copy(zero_v, out_hbm.at[c].at[pl.ds(r0, CH)])

    return deg_kernel


@functools.lru_cache(maxsize=None)
def _make_agg_kernel(n, n_pad, k, d):
    rps = n_pad // NS
    zb = rps // CH

    @functools.partial(
        pl.kernel,
        mesh=_MESH,
        out_type=jax.ShapeDtypeStruct((NC, n_pad, d), jnp.float32),
        scratch_types=[
            pltpu.VMEM((k, CH), jnp.int32),
            pltpu.VMEM((k, CH), jnp.int32),
            pltpu.VMEM((CH, d), jnp.float32),
            pltpu.VMEM((CH, d), jnp.float32),
            pltpu.VMEM_SHARED((n_pad, d), jnp.float32),
            pltpu.SemaphoreType.DMA,
        ],
    )
    def agg_kernel(g_hbm, src_hbm, dst_hbm, zeros_hbm, out_hbm,
                   src_v, dst_v, rows_v, zero_v, shared, sem):
        c = lax.axis_index("c")
        s = lax.axis_index("s")
        wid = c * NS + s
        pltpu.sync_copy(src_hbm.at[wid], src_v)
        pltpu.sync_copy(dst_hbm.at[wid], dst_v)
        pltpu.sync_copy(zeros_hbm, zero_v)

        @pl.loop(0, zb)
        def _(z):
            pltpu.sync_copy(zero_v, shared.at[pl.ds(s * rps + z * CH, CH)])

        plsc.subcore_barrier()

        @pl.loop(0, k)
        def _(j):
            pltpu.async_copy(g_hbm.at[src_v.at[j]], rows_v, sem).wait()
            pltpu.sync_copy(rows_v, shared.at[dst_v.at[j]], add=True)

        plsc.subcore_barrier()

        @pl.loop(0, zb)
        def _(z):
            r0 = s * rps + z * CH
            pltpu.sync_copy(shared.at[pl.ds(r0, CH)], rows_v)
            pltpu.sync_copy(rows_v, out_hbm.at[c].at[pl.ds(r0, CH)])

    return agg_kernel


def _mm_scale_body(a_ref, w_ref, deg_ref, o_ref):
    dinv = lax.rsqrt(deg_ref[...])
    o_ref[...] = jnp.dot(a_ref[...], w_ref[...],
                         preferred_element_type=jnp.float32) * dinv


def _mm_scale(a, w, deg, r):
    n, d = a.shape
    return pl.pallas_call(
        _mm_scale_body,
        grid=(n // r,),
        in_specs=[
            pl.BlockSpec((r, d), lambda i: (i, 0)),
            pl.BlockSpec((d, d), lambda i: (0, 0)),
            pl.BlockSpec((r, 1), lambda i: (i, 0)),
        ],
        out_specs=pl.BlockSpec((r, d), lambda i: (i, 0)),
        out_shape=jax.ShapeDtypeStruct((n, d), jnp.float32),
    )(a, w, deg)


def _combine_body(agg_ref, g_ref, deg_ref, b_ref, o_ref):
    dinv = lax.rsqrt(deg_ref[...])
    s = (agg_ref[0] + agg_ref[1] + g_ref[...]) * dinv + b_ref[...]
    o_ref[...] = jnp.maximum(s, 0.0)


def _combine(agg, g, deg, b, r):
    n, d = g.shape
    return pl.pallas_call(
        _combine_body,
        grid=(n // r,),
        in_specs=[
            pl.BlockSpec((NC, r, d), lambda i: (0, i, 0)),
            pl.BlockSpec((r, d), lambda i: (i, 0)),
            pl.BlockSpec((r, 1), lambda i: (i, 0)),
            pl.BlockSpec((1, d), lambda i: (0, 0)),
        ],
        out_specs=pl.BlockSpec((r, d), lambda i: (i, 0)),
        out_shape=jax.ShapeDtypeStruct((n, d), jnp.float32),
    )(agg, g, deg, b)


def _final_body(a_ref, w_ref, b_ref, o_ref):
    o_ref[...] = jnp.dot(a_ref[...], w_ref[...],
                         preferred_element_type=jnp.float32) + b_ref[...]


def _final(a, w, b, r):
    n, d = a.shape
    return pl.pallas_call(
        _final_body,
        grid=(n // r,),
        in_specs=[
            pl.BlockSpec((r, d), lambda i: (i, 0)),
            pl.BlockSpec((d, 1), lambda i: (0, 0)),
            pl.BlockSpec((1, 1), lambda i: (0, 0)),
        ],
        out_specs=pl.BlockSpec((r, 1), lambda i: (i, 0)),
        out_shape=jax.ShapeDtypeStruct((n, 1), jnp.float32),
    )(a, w, b)


def kernel(num_layers, x, edge_index, W0, b0, W1, b1, W2, b2, W3, b3, W4, b4,
           Wout, bout):
    n, d = x.shape
    e = edge_index.shape[1]
    n_pad = _ceil_to(n + 1, NS * CH)
    epw = _ceil_to(-(-e // NW), CH)
    k = epw // CH
    e_pad = epw * NW
    r = 1000

    src = edge_index[0].astype(jnp.int32)
    dst = edge_index[1].astype(jnp.int32)
    npad_e = e_pad - e
    ar = jnp.arange(npad_e, dtype=jnp.int32)
    src_p = jnp.concatenate([src, ar % n]).reshape(NW, k, CH)
    dst_p = jnp.concatenate([dst, n + ar % (n_pad - n)]).reshape(NW, k, CH)

    ones16 = jnp.ones((CH, DW), jnp.float32)
    zeros16 = jnp.zeros((CH, DW), jnp.float32)
    zerosd = jnp.zeros((CH, d), jnp.float32)

    degp = _make_deg_kernel(n_pad, k)(dst_p, ones16, zeros16)
    deg = (degp[0, :n, 0] + degp[1, :n, 0] + 1.0).reshape(n, 1)

    agg_kernel = _make_agg_kernel(n, n_pad, k, d)

    def conv(a, w, b):
        g = _mm_scale(a, w, deg, r)
        agg = agg_kernel(g, src_p, dst_p, zerosd)
        return _combine(agg, g, deg, b.reshape(1, d), r)

    h = conv(x, W0, b0)
    h = conv(h, W1, b1)
    h = jnp.where(num_layers > 1, conv(h, W2, b2), h)
    h = jnp.where(num_layers > 2, conv(h, W3, b3), h)
    h = jnp.where(num_layers > 3, conv(h, W4, b4), h)
    return _final(h, Wout, bout.reshape(1, 1), r)


# speedup vs baseline: 12.6667x; 12.6667x over previous
import functools

import jax
import jax.numpy as jnp
from jax import lax
from jax.experimental import pallas as pl
from jax.experimental.pallas import tpu as pltpu
from jax.experimental.pallas import tpu_sc as plsc

NC = 2
NS = 16
NW = NC * NS
CH = 128
SB = 8

_MESH = plsc.VectorSubcoreMesh(core_axis_name="c", subcore_axis_name="s")


def _ceil_to(v, m):
    return -(-v // m) * m


@functools.lru_cache(maxsize=None)
def _make_agg_kernel(n, n_pad, k, d):
    rps = n_pad // NS
    zb = rps // CH

    @functools.partial(
        pl.kernel,
        mesh=_MESH,
        out_type=jax.ShapeDtypeStruct((NC, n_pad, d), jnp.float32),
        scratch_types=[
            pltpu.VMEM((SB, CH), jnp.int32),
            pltpu.VMEM((SB, CH), jnp.int32),
            pltpu.VMEM((CH, d), jnp.float32),
            pltpu.VMEM((CH, d), jnp.float32),
            pltpu.VMEM_SHARED((n_pad, d), jnp.float32),
            pltpu.SemaphoreType.DMA,
        ],
    )
    def agg_kernel(g_hbm, src_hbm, dst_hbm, zeros_hbm, out_hbm,
                   src_v, dst_v, rows_v, zero_v, shared, sem):
        c = lax.axis_index("c")
        s = lax.axis_index("s")
        wid = c * NS + s
        pltpu.sync_copy(zeros_hbm, zero_v)

        @pl.loop(0, zb)
        def _(z):
            pltpu.sync_copy(zero_v, shared.at[pl.ds(s * rps + z * CH, CH)])

        plsc.subcore_barrier()

        @pl.loop(0, k // SB)
        def _(jb):
            pltpu.sync_copy(src_hbm.at[wid].at[pl.ds(jb * SB, SB)], src_v)
            pltpu.sync_copy(dst_hbm.at[wid].at[pl.ds(jb * SB, SB)], dst_v)

            @pl.loop(0, SB)
            def _(jj):
                pltpu.async_copy(g_hbm.at[src_v.at[jj]], rows_v, sem).wait()
                pltpu.sync_copy(rows_v, shared.at[dst_v.at[jj]], add=True)

        plsc.subcore_barrier()

        @pl.loop(0, zb)
        def _(z):
            r0 = s * rps + z * CH
            pltpu.sync_copy(shared.at[pl.ds(r0, CH)], rows_v)
            pltpu.sync_copy(rows_v, out_hbm.at[c].at[pl.ds(r0, CH)])

    return agg_kernel


def _mm_scale_body(a_ref, w_ref, deg_ref, o_ref):
    dinv = lax.rsqrt(deg_ref[...])
    o_ref[...] = jnp.dot(a_ref[...], w_ref[...],
                         preferred_element_type=jnp.float32) * dinv


def _mm_scale(a, w, deg, r):
    n, d = a.shape
    return pl.pallas_call(
        _mm_scale_body,
        grid=(n // r,),
        in_specs=[
            pl.BlockSpec((r, d), lambda i: (i, 0)),
            pl.BlockSpec((d, d), lambda i: (0, 0)),
            pl.BlockSpec((r, 1), lambda i: (i, 0)),
        ],
        out_specs=pl.BlockSpec((r, d), lambda i: (i, 0)),
        out_shape=jax.ShapeDtypeStruct((n, d), jnp.float32),
    )(a, w, deg)


def _combine_body(agg_ref, g_ref, deg_ref, b_ref, o_ref):
    dinv = lax.rsqrt(deg_ref[...])
    s = (agg_ref[0] + agg_ref[1] + g_ref[...]) * dinv + b_ref[...]
    o_ref[...] = jnp.maximum(s, 0.0)


def _combine(agg, g, deg, b, r):
    n, d = g.shape
    return pl.pallas_call(
        _combine_body,
        grid=(n // r,),
        in_specs=[
            pl.BlockSpec((NC, r, d), lambda i: (0, i, 0)),
            pl.BlockSpec((r, d), lambda i: (i, 0)),
            pl.BlockSpec((r, 1), lambda i: (i, 0)),
            pl.BlockSpec((1, d), lambda i: (0, 0)),
        ],
        out_specs=pl.BlockSpec((r, d), lambda i: (i, 0)),
        out_shape=jax.ShapeDtypeStruct((n, d), jnp.float32),
    )(agg, g, deg, b)


def _final_body(a_ref, w_ref, b_ref, o_ref):
    o_ref[...] = jnp.dot(a_ref[...], w_ref[...],
                         preferred_element_type=jnp.float32) + b_ref[...]


def _final(a, w, b, r):
    n, d = a.shape
    return pl.pallas_call(
        _final_body,
        grid=(n // r,),
        in_specs=[
            pl.BlockSpec((r, d), lambda i: (i, 0)),
            pl.BlockSpec((d, 1), lambda i: (0, 0)),
            pl.BlockSpec((1, 1), lambda i: (0, 0)),
        ],
        out_specs=pl.BlockSpec((r, 1), lambda i: (i, 0)),
        out_shape=jax.ShapeDtypeStruct((n, 1), jnp.float32),
    )(a, w, b)


def kernel(num_layers, x, edge_index, W0, b0, W1, b1, W2, b2, W3, b3, W4, b4,
           Wout, bout):
    n, d = x.shape
    e = edge_index.shape[1]
    n_pad = _ceil_to(n + 1, NS * CH)
    epw = _ceil_to(-(-e // NW), SB * CH)
    k = epw // CH
    e_pad = epw * NW
    r = 1000

    src = edge_index[0].astype(jnp.int32)
    dst = edge_index[1].astype(jnp.int32)
    npad_e = e_pad - e
    ar = jnp.arange(npad_e, dtype=jnp.int32)
    src_p = jnp.concatenate([src, ar % n]).reshape(NW, k, CH)
    dst_p = jnp.concatenate([dst, n + ar % (n_pad - n)]).reshape(NW, k, CH)

    zerosd = jnp.zeros((CH, d), jnp.float32)

    agg_kernel = _make_agg_kernel(n, n_pad, k, d)

    degp = agg_kernel(jnp.ones((n, d), jnp.float32), src_p, dst_p, zerosd)
    deg = (degp[0, :n, 0] + degp[1, :n, 0] + 1.0).reshape(n, 1)

    def conv(a, w, b):
        g = _mm_scale(a, w, deg, r)
        agg = agg_kernel(g, src_p, dst_p, zerosd)
        return _combine(agg, g, deg, b.reshape(1, d), r)

    h = conv(x, W0, b0)
    h = conv(h, W1, b1)
    h = jnp.where(num_layers > 1, conv(h, W2, b2), h)
    h = jnp.where(num_layers > 2, conv(h, W3, b3), h)
    h = jnp.where(num_layers > 3, conv(h, W4, b4), h)
    return _final(h, Wout, bout.reshape(1, 1), r)


# speedup vs baseline: 15.7907x; 1.2466x over previous
import functools

import jax
import jax.numpy as jnp
from jax import lax
from jax.experimental import pallas as pl
from jax.experimental.pallas import tpu as pltpu
from jax.experimental.pallas import tpu_sc as plsc

NC = 2
NS = 16
NW = NC * NS
CH = 128
SB = 16

_MESH = plsc.VectorSubcoreMesh(core_axis_name="c", subcore_axis_name="s")


def _ceil_to(v, m):
    return -(-v // m) * m


@functools.lru_cache(maxsize=None)
def _make_agg_kernel(n, n_pad, k, d):
    rps = n_pad // NS
    zb = rps // CH

    @functools.partial(
        pl.kernel,
        mesh=_MESH,
        out_type=jax.ShapeDtypeStruct((NC, n_pad, d), jnp.float32),
        scratch_types=[
            pltpu.VMEM((SB, CH), jnp.int32),
            pltpu.VMEM((SB, CH), jnp.int32),
            pltpu.VMEM((CH, d), jnp.float32),
            pltpu.VMEM((CH, d), jnp.float32),
            pltpu.VMEM_SHARED((n_pad, d), jnp.float32),
            pltpu.SemaphoreType.DMA,
            pltpu.SemaphoreType.DMA,
        ],
    )
    def agg_kernel(g_hbm, src_hbm, dst_hbm, zeros_hbm, out_hbm,
                   src_v, dst_v, rows_a, rows_b, shared, gsem, ssem):
        c = lax.axis_index("c")
        s = lax.axis_index("s")
        wid = c * NS + s

        @pl.loop(0, zb)
        def _(z):
            pltpu.sync_copy(zeros_hbm, shared.at[pl.ds(s * rps + z * CH, CH)])

        plsc.subcore_barrier()

        @pl.loop(0, k // SB)
        def _(jb):
            pltpu.sync_copy(src_hbm.at[wid].at[pl.ds(jb * SB, SB)], src_v)
            pltpu.sync_copy(dst_hbm.at[wid].at[pl.ds(jb * SB, SB)], dst_v)
            bufs = (rows_a, rows_b)
            gh = {0: pltpu.async_copy(g_hbm.at[src_v.at[0]], bufs[0], gsem)}
            sh = {}
            for j in range(SB):
                cur = bufs[j % 2]
                gh[j].wait()
                if j >= 1:
                    sh[j - 1].wait()
                if j + 1 < SB:
                    gh[j + 1] = pltpu.async_copy(
                        g_hbm.at[src_v.at[j + 1]], bufs[(j + 1) % 2], gsem)
                sh[j] = pltpu.async_copy(
                    cur, shared.at[dst_v.at[j]], ssem, add=True)
            sh[SB - 1].wait()

        plsc.subcore_barrier()

        @pl.loop(0, zb)
        def _(z):
            r0 = s * rps + z * CH
            pltpu.sync_copy(shared.at[pl.ds(r0, CH)], out_hbm.at[c].at[pl.ds(r0, CH)])

    return agg_kernel


def _mm_scale_body(a_ref, w_ref, deg_ref, o_ref):
    dinv = lax.rsqrt(deg_ref[...])
    o_ref[...] = jnp.dot(a_ref[...], w_ref[...],
                         preferred_element_type=jnp.float32) * dinv


def _mm_scale(a, w, deg, r):
    n, d = a.shape
    return pl.pallas_call(
        _mm_scale_body,
        grid=(n // r,),
        in_specs=[
            pl.BlockSpec((r, d), lambda i: (i, 0)),
            pl.BlockSpec((d, d), lambda i: (0, 0)),
            pl.BlockSpec((r, 1), lambda i: (i, 0)),
        ],
        out_specs=pl.BlockSpec((r, d), lambda i: (i, 0)),
        out_shape=jax.ShapeDtypeStruct((n, d), jnp.float32),
    )(a, w, deg)


def _combine_body(agg_ref, g_ref, deg_ref, b_ref, o_ref):
    dinv = lax.rsqrt(deg_ref[...])
    s = (agg_ref[0] + agg_ref[1] + g_ref[...]) * dinv + b_ref[...]
    o_ref[...] = jnp.maximum(s, 0.0)


def _combine(agg, g, deg, b, r):
    n, d = g.shape
    return pl.pallas_call(
        _combine_body,
        grid=(n // r,),
        in_specs=[
            pl.BlockSpec((NC, r, d), lambda i: (0, i, 0)),
            pl.BlockSpec((r, d), lambda i: (i, 0)),
            pl.BlockSpec((r, 1), lambda i: (i, 0)),
            pl.BlockSpec((1, d), lambda i: (0, 0)),
        ],
        out_specs=pl.BlockSpec((r, d), lambda i: (i, 0)),
        out_shape=jax.ShapeDtypeStruct((n, d), jnp.float32),
    )(agg, g, deg, b)


def _final_body(a_ref, w_ref, b_ref, o_ref):
    o_ref[...] = jnp.dot(a_ref[...], w_ref[...],
                         preferred_element_type=jnp.float32) + b_ref[...]


def _final(a, w, b, r):
    n, d = a.shape
    return pl.pallas_call(
        _final_body,
        grid=(n // r,),
        in_specs=[
            pl.BlockSpec((r, d), lambda i: (i, 0)),
            pl.BlockSpec((d, 1), lambda i: (0, 0)),
            pl.BlockSpec((1, 1), lambda i: (0, 0)),
        ],
        out_specs=pl.BlockSpec((r, 1), lambda i: (i, 0)),
        out_shape=jax.ShapeDtypeStruct((n, 1), jnp.float32),
    )(a, w, b)


def kernel(num_layers, x, edge_index, W0, b0, W1, b1, W2, b2, W3, b3, W4, b4,
           Wout, bout):
    n, d = x.shape
    e = edge_index.shape[1]
    n_pad = _ceil_to(n + 1, NS * CH)
    epw = _ceil_to(-(-e // NW), SB * CH)
    k = epw // CH
    e_pad = epw * NW
    r = 1000

    src = edge_index[0].astype(jnp.int32)
    dst = edge_index[1].astype(jnp.int32)
    npad_e = e_pad - e
    ar = jnp.arange(npad_e, dtype=jnp.int32)
    src_p = jnp.concatenate([src, ar % n]).reshape(NW, k, CH)
    dst_p = jnp.concatenate([dst, n + ar % (n_pad - n)]).reshape(NW, k, CH)

    zerosd = jnp.zeros((CH, d), jnp.float32)

    agg_kernel = _make_agg_kernel(n, n_pad, k, d)

    degp = agg_kernel(jnp.ones((n, d), jnp.float32), src_p, dst_p, zerosd)
    deg = (degp[0, :n, 0] + degp[1, :n, 0] + 1.0).reshape(n, 1)

    def conv(a, w, b):
        g = _mm_scale(a, w, deg, r)
        agg = agg_kernel(g, src_p, dst_p, zerosd)
        return _combine(agg, g, deg, b.reshape(1, d), r)

    h = conv(x, W0, b0)
    h = conv(h, W1, b1)
    h = jnp.where(num_layers > 1, conv(h, W2, b2), h)
    h = jnp.where(num_layers > 2, conv(h, W3, b3), h)
    h = jnp.where(num_layers > 3, conv(h, W4, b4), h)
    return _final(h, Wout, bout.reshape(1, 1), r)


# speedup vs baseline: 17.8967x; 1.1334x over previous
import functools

import jax
import jax.numpy as jnp
from jax import lax
from jax.experimental import pallas as pl
from jax.experimental.pallas import tpu as pltpu
from jax.experimental.pallas import tpu_sc as plsc

NC = 2
NS = 16
NW = NC * NS
CH = 128
SB = 16

_MESH = plsc.VectorSubcoreMesh(core_axis_name="c", subcore_axis_name="s")


def _ceil_to(v, m):
    return -(-v // m) * m


@functools.lru_cache(maxsize=None)
def _make_agg_kernel(n, n_pad, k, d):
    rps = n_pad // NS
    zb = rps // CH

    @functools.partial(
        pl.kernel,
        mesh=_MESH,
        out_type=jax.ShapeDtypeStruct((NC, n_pad, d), jnp.float32),
        scratch_types=[
            pltpu.VMEM((SB, CH), jnp.int32),
            pltpu.VMEM((SB, CH), jnp.int32),
            pltpu.VMEM((CH, d), jnp.float32),
            pltpu.VMEM((CH, d), jnp.float32),
            pltpu.VMEM_SHARED((n_pad, d), jnp.float32),
            pltpu.SemaphoreType.DMA,
            pltpu.SemaphoreType.DMA,
        ],
    )
    def agg_kernel(g_hbm, src_hbm, dst_hbm, zeros_hbm, out_hbm,
                   src_v, dst_v, rows_a, rows_b, shared, gsem, ssem):
        c = lax.axis_index("c")
        s = lax.axis_index("s")
        wid = c * NS + s

        @pl.loop(0, zb)
        def _(z):
            pltpu.sync_copy(zeros_hbm, shared.at[pl.ds(s * rps + z * CH, CH)])

        plsc.subcore_barrier()

        @pl.loop(0, k // SB)
        def _(jb):
            pltpu.sync_copy(src_hbm.at[wid].at[pl.ds(jb * SB, SB)], src_v)
            pltpu.sync_copy(dst_hbm.at[wid].at[pl.ds(jb * SB, SB)], dst_v)
            bufs = (rows_a, rows_b)
            gh = {0: pltpu.async_copy(g_hbm.at[src_v.at[0]], bufs[0], gsem)}
            sh = {}
            for j in range(SB):
                cur = bufs[j % 2]
                gh[j].wait()
                if j >= 1:
                    sh[j - 1].wait()
                if j + 1 < SB:
                    gh[j + 1] = pltpu.async_copy(
                        g_hbm.at[src_v.at[j + 1]], bufs[(j + 1) % 2], gsem)
                sh[j] = pltpu.async_copy(
                    cur, shared.at[dst_v.at[j]], ssem, add=True)
            sh[SB - 1].wait()

        plsc.subcore_barrier()

        @pl.loop(0, zb)
        def _(z):
            r0 = s * rps + z * CH
            pltpu.sync_copy(shared.at[pl.ds(r0, CH)], out_hbm.at[c].at[pl.ds(r0, CH)])

    return agg_kernel


@functools.lru_cache(maxsize=None)
def _make_deg_kernel(n_pad, k, d):
    rps = n_pad // NS
    zb = rps // CH

    @functools.partial(
        pl.kernel,
        mesh=_MESH,
        out_type=jax.ShapeDtypeStruct((NC, n_pad, d), jnp.float32),
        scratch_types=[
            pltpu.VMEM((SB, CH), jnp.int32),
            pltpu.VMEM((CH, d), jnp.float32),
            pltpu.VMEM_SHARED((n_pad, d), jnp.float32),
            pltpu.SemaphoreType.DMA,
        ],
    )
    def deg_kernel(dst_hbm, ones_hbm, zeros_hbm, out_hbm,
                   dst_v, ones_v, shared, ssem):
        c = lax.axis_index("c")
        s = lax.axis_index("s")
        wid = c * NS + s
        pltpu.sync_copy(ones_hbm, ones_v)

        @pl.loop(0, zb)
        def _(z):
            pltpu.sync_copy(zeros_hbm, shared.at[pl.ds(s * rps + z * CH, CH)])

        plsc.subcore_barrier()

        @pl.loop(0, k // SB)
        def _(jb):
            pltpu.sync_copy(dst_hbm.at[wid].at[pl.ds(jb * SB, SB)], dst_v)
            sh = {}
            for j in range(SB):
                sh[j] = pltpu.async_copy(
                    ones_v, shared.at[dst_v.at[j]], ssem, add=True)
                if j >= 1:
                    sh[j - 1].wait()
            sh[SB - 1].wait()

        plsc.subcore_barrier()

        @pl.loop(0, zb)
        def _(z):
            r0 = s * rps + z * CH
            pltpu.sync_copy(shared.at[pl.ds(r0, CH)], out_hbm.at[c].at[pl.ds(r0, CH)])

    return deg_kernel


def _mm_scale_body(a_ref, w_ref, deg_ref, o_ref):
    dinv = lax.rsqrt(deg_ref[...])
    o_ref[...] = jnp.dot(a_ref[...], w_ref[...],
                         preferred_element_type=jnp.float32) * dinv


def _mm_scale(a, w, deg, r):
    n, d = a.shape
    return pl.pallas_call(
        _mm_scale_body,
        grid=(n // r,),
        in_specs=[
            pl.BlockSpec((r, d), lambda i: (i, 0)),
            pl.BlockSpec((d, d), lambda i: (0, 0)),
            pl.BlockSpec((r, 1), lambda i: (i, 0)),
        ],
        out_specs=pl.BlockSpec((r, d), lambda i: (i, 0)),
        out_shape=jax.ShapeDtypeStruct((n, d), jnp.float32),
    )(a, w, deg)


def _relu_comb(agg_ref, g_ref, deg_ref, b_ref):
    dinv = lax.rsqrt(deg_ref[...])
    return dinv, jnp.maximum(
        (agg_ref[0] + agg_ref[1] + g_ref[...]) * dinv + b_ref[...], 0.0)


def _fused_mid_body(agg_ref, g_ref, deg_ref, b_ref, w_ref, o_ref):
    dinv, t = _relu_comb(agg_ref, g_ref, deg_ref, b_ref)
    o_ref[...] = jnp.dot(t, w_ref[...],
                         preferred_element_type=jnp.float32) * dinv


def _fused_keep_body(agg_ref, g_ref, deg_ref, b_ref, w_ref, h_ref, o_ref):
    dinv, t = _relu_comb(agg_ref, g_ref, deg_ref, b_ref)
    h_ref[...] = t
    o_ref[...] = jnp.dot(t, w_ref[...],
                         preferred_element_type=jnp.float32) * dinv


def _fused_cond_body(thresh, agg_ref, g_ref, deg_ref, b_ref, hp_ref, nl_ref,
                     w_ref, h_ref, o_ref):
    dinv, t = _relu_comb(agg_ref, g_ref, deg_ref, b_ref)
    t = jnp.where(nl_ref[0, 0] > thresh, t, hp_ref[...])
    h_ref[...] = t
    o_ref[...] = jnp.dot(t, w_ref[...],
                         preferred_element_type=jnp.float32) * dinv


def _fused_out_body(agg_ref, g_ref, deg_ref, b_ref, hp_ref, nl_ref,
                    w_ref, bo_ref, o_ref):
    _, t = _relu_comb(agg_ref, g_ref, deg_ref, b_ref)
    t = jnp.where(nl_ref[0, 0] > 3, t, hp_ref[...])
    o_ref[...] = jnp.dot(t, w_ref[...],
                         preferred_element_type=jnp.float32) + bo_ref[...]


def _spec_base(r, d, n_pad_unused=None):
    return [
        pl.BlockSpec((NC, r, d), lambda i: (0, i, 0)),
        pl.BlockSpec((r, d), lambda i: (i, 0)),
        pl.BlockSpec((r, 1), lambda i: (i, 0)),
        pl.BlockSpec((1, d), lambda i: (0, 0)),
    ]


def _fused_mid(agg, g, deg, b, w, r):
    n, d = g.shape
    return pl.pallas_call(
        _fused_mid_body,
        grid=(n // r,),
        in_specs=_spec_base(r, d) + [pl.BlockSpec((d, d), lambda i: (0, 0))],
        out_specs=pl.BlockSpec((r, d), lambda i: (i, 0)),
        out_shape=jax.ShapeDtypeStruct((n, d), jnp.float32),
    )(agg, g, deg, b.reshape(1, d), w)


def _fused_keep(agg, g, deg, b, w, r):
    n, d = g.shape
    return pl.pallas_call(
        _fused_keep_body,
        grid=(n // r,),
        in_specs=_spec_base(r, d) + [pl.BlockSpec((d, d), lambda i: (0, 0))],
        out_specs=[pl.BlockSpec((r, d), lambda i: (i, 0)),
                   pl.BlockSpec((r, d), lambda i: (i, 0))],
        out_shape=[jax.ShapeDtypeStruct((n, d), jnp.float32),
                   jax.ShapeDtypeStruct((n, d), jnp.float32)],
    )(agg, g, deg, b.reshape(1, d), w)


def _fused_cond(agg, g, deg, b, hp, nl, w, r, thresh):
    n, d = g.shape
    return pl.pallas_call(
        functools.partial(_fused_cond_body, thresh),
        grid=(n // r,),
        in_specs=_spec_base(r, d) + [
            pl.BlockSpec((r, d), lambda i: (i, 0)),
            pl.BlockSpec((1, 1), lambda i: (0, 0)),
            pl.BlockSpec((d, d), lambda i: (0, 0)),
        ],
        out_specs=[pl.BlockSpec((r, d), lambda i: (i, 0)),
                   pl.BlockSpec((r, d), lambda i: (i, 0))],
        out_shape=[jax.ShapeDtypeStruct((n, d), jnp.float32),
                   jax.ShapeDtypeStruct((n, d), jnp.float32)],
    )(agg, g, deg, b.reshape(1, d), hp, nl, w)


def _fused_out(agg, g, deg, b, hp, nl, w, bo, r):
    n, d = g.shape
    return pl.pallas_call(
        _fused_out_body,
        grid=(n // r,),
        in_specs=_spec_base(r, d) + [
            pl.BlockSpec((r, d), lambda i: (i, 0)),
            pl.BlockSpec((1, 1), lambda i: (0, 0)),
            pl.BlockSpec((d, 1), lambda i: (0, 0)),
            pl.BlockSpec((1, 1), lambda i: (0, 0)),
        ],
        out_specs=pl.BlockSpec((r, 1), lambda i: (i, 0)),
        out_shape=jax.ShapeDtypeStruct((n, 1), jnp.float32),
    )(agg, g, deg, b.reshape(1, d), hp, nl, w, bo.reshape(1, 1))


def kernel(num_layers, x, edge_index, W0, b0, W1, b1, W2, b2, W3, b3, W4, b4,
           Wout, bout):
    n, d = x.shape
    e = edge_index.shape[1]
    n_pad = _ceil_to(n + 1, NS * CH)
    epw = _ceil_to(-(-e // NW), SB * CH)
    k = epw // CH
    e_pad = epw * NW
    r = 1000

    src = edge_index[0].astype(jnp.int32)
    dst = edge_index[1].astype(jnp.int32)
    npad_e = e_pad - e
    ar = jnp.arange(npad_e, dtype=jnp.int32)
    src_p = jnp.concatenate([src, ar % n]).reshape(NW, k, CH)
    dst_p = jnp.concatenate([dst, n + ar % (n_pad - n)]).reshape(NW, k, CH)

    zerosd = jnp.zeros((CH, d), jnp.float32)
    onesd = jnp.ones((CH, d), jnp.float32)
    nl = jnp.full((1, 1), num_layers, jnp.int32)

    degp = _make_deg_kernel(n_pad, k, d)(dst_p, onesd, zerosd)
    deg = (degp[0, :n, 0] + degp[1, :n, 0] + 1.0).reshape(n, 1)

    agg = _make_agg_kernel(n, n_pad, k, d)

    def do_agg(g):
        return agg(g, src_p, dst_p, zerosd)

    g0 = _mm_scale(x, W0, deg, r)
    a0 = do_agg(g0)
    g1 = _fused_mid(a0, g0, deg, b0, W1, r)
    a1 = do_agg(g1)
    h2, g2 = _fused_keep(a1, g1, deg, b1, W2, r)
    a2 = do_agg(g2)
    h3, g3 = _fused_cond(a2, g2, deg, b2, h2, nl, W3, r, 1)
    a3 = do_agg(g3)
    h4, g4 = _fused_cond(a3, g3, deg, b3, h3, nl, W4, r, 2)
    a4 = do_agg(g4)
    return _fused_out(a4, g4, deg, b4, h4, nl, Wout, bout, r)


# speedup vs baseline: 18.0340x; 1.0077x over previous
import functools

import jax
import jax.numpy as jnp
from jax import lax
from jax.experimental import pallas as pl
from jax.experimental.pallas import tpu as pltpu
from jax.experimental.pallas import tpu_sc as plsc

NC = 2
NS = 16
NW = NC * NS
CH = 128
SB = 16

_MESH = plsc.VectorSubcoreMesh(core_axis_name="c", subcore_axis_name="s")


def _ceil_to(v, m):
    return -(-v // m) * m


@functools.lru_cache(maxsize=None)
def _make_agg_kernel(n, n_pad, k, d):
    rps = n_pad // NS
    zb = rps // CH

    @functools.partial(
        pl.kernel,
        mesh=_MESH,
        out_type=jax.ShapeDtypeStruct((NC, n_pad, d), jnp.float32),
        scratch_types=[
            pltpu.VMEM((SB, CH), jnp.int32),
            pltpu.VMEM((SB, CH), jnp.int32),
            pltpu.VMEM((CH, d), jnp.float32),
            pltpu.VMEM((CH, d), jnp.float32),
            pltpu.VMEM_SHARED((n_pad, d), jnp.float32),
            pltpu.SemaphoreType.DMA,
            pltpu.SemaphoreType.DMA,
        ],
    )
    def agg_kernel(g_hbm, src_hbm, dst_hbm, zeros_hbm, out_hbm,
                   src_v, dst_v, rows_a, rows_b, shared, gsem, ssem):
        c = lax.axis_index("c")
        s = lax.axis_index("s")
        wid = c * NS + s

        @pl.loop(0, zb)
        def _(z):
            pltpu.sync_copy(zeros_hbm, shared.at[pl.ds(s * rps + z * CH, CH)])

        plsc.subcore_barrier()

        @pl.loop(0, k // SB)
        def _(jb):
            pltpu.sync_copy(src_hbm.at[wid].at[pl.ds(jb * SB, SB)], src_v)
            pltpu.sync_copy(dst_hbm.at[wid].at[pl.ds(jb * SB, SB)], dst_v)
            bufs = (rows_a, rows_b)
            gh = {0: pltpu.async_copy(g_hbm.at[src_v.at[0]], bufs[0], gsem)}
            sh = {}
            for j in range(SB):
                cur = bufs[j % 2]
                gh[j].wait()
                if j >= 1:
                    sh[j - 1].wait()
                if j + 1 < SB:
                    gh[j + 1] = pltpu.async_copy(
                        g_hbm.at[src_v.at[j + 1]], bufs[(j + 1) % 2], gsem)
                sh[j] = pltpu.async_copy(
                    cur, shared.at[dst_v.at[j]], ssem, add=True)
            sh[SB - 1].wait()

        plsc.subcore_barrier()

        @pl.loop(0, zb)
        def _(z):
            r0 = s * rps + z * CH
            pltpu.sync_copy(shared.at[pl.ds(r0, CH)], out_hbm.at[c].at[pl.ds(r0, CH)])

    return agg_kernel


@functools.lru_cache(maxsize=None)
def _make_deg_kernel(n_pad, k, d):
    rps = n_pad // NS
    zb = rps // CH

    @functools.partial(
        pl.kernel,
        mesh=_MESH,
        out_type=jax.ShapeDtypeStruct((NC, n_pad, d), jnp.float32),
        scratch_types=[
            pltpu.VMEM((SB, CH), jnp.int32),
            pltpu.VMEM((CH, d), jnp.float32),
            pltpu.VMEM_SHARED((n_pad, d), jnp.float32),
            pltpu.SemaphoreType.DMA,
        ],
    )
    def deg_kernel(dst_hbm, ones_hbm, zeros_hbm, out_hbm,
                   dst_v, ones_v, shared, ssem):
        c = lax.axis_index("c")
        s = lax.axis_index("s")
        wid = c * NS + s
        pltpu.sync_copy(ones_hbm, ones_v)

        @pl.loop(0, zb)
        def _(z):
            pltpu.sync_copy(zeros_hbm, shared.at[pl.ds(s * rps + z * CH, CH)])

        plsc.subcore_barrier()

        @pl.loop(0, k // SB)
        def _(jb):
            pltpu.sync_copy(dst_hbm.at[wid].at[pl.ds(jb * SB, SB)], dst_v)
            sh = {}
            for j in range(SB):
                sh[j] = pltpu.async_copy(
                    ones_v, shared.at[dst_v.at[j]], ssem, add=True)
                if j >= 1:
                    sh[j - 1].wait()
            sh[SB - 1].wait()

        plsc.subcore_barrier()

        @pl.loop(0, zb)
        def _(z):
            r0 = s * rps + z * CH
            pltpu.sync_copy(shared.at[pl.ds(r0, CH)], out_hbm.at[c].at[pl.ds(r0, CH)])

    return deg_kernel


def _dot(a, b):
    return jnp.dot(a, b, preferred_element_type=jnp.float32)


def _mm_body(a_ref, w_ref, o_ref):
    o_ref[...] = _dot(a_ref[...], w_ref[...])


def _mm(a, w, r):
    n, d = a.shape
    return pl.pallas_call(
        _mm_body,
        grid=(n // r,),
        in_specs=[
            pl.BlockSpec((r, d), lambda i: (i, 0)),
            pl.BlockSpec((d, d), lambda i: (0, 0)),
        ],
        out_specs=pl.BlockSpec((r, d), lambda i: (i, 0)),
        out_shape=jax.ShapeDtypeStruct((n, d), jnp.float32),
    )(a, w)


def _scale0_body(u_ref, degp_ref, o_ref, deg_ref):
    deg = degp_ref[0, :, 0:1] + degp_ref[1, :, 0:1] + 1.0
    deg_ref[...] = deg
    o_ref[...] = u_ref[...] * lax.rsqrt(deg)


def _scale0(u, degp, r):
    n, d = u.shape
    return pl.pallas_call(
        _scale0_body,
        grid=(n // r,),
        in_specs=[
            pl.BlockSpec((r, d), lambda i: (i, 0)),
            pl.BlockSpec((NC, r, d), lambda i: (0, i, 0)),
        ],
        out_specs=[pl.BlockSpec((r, d), lambda i: (i, 0)),
                   pl.BlockSpec((r, 1), lambda i: (i, 0))],
        out_shape=[jax.ShapeDtypeStruct((n, d), jnp.float32),
                   jax.ShapeDtypeStruct((n, 1), jnp.float32)],
    )(u, degp)


def _relu_comb(agg_ref, g_ref, deg_ref, b_ref):
    dinv = lax.rsqrt(deg_ref[...])
    return dinv, jnp.maximum(
        (agg_ref[0] + agg_ref[1] + g_ref[...]) * dinv + b_ref[...], 0.0)


def _fused_mid_body(agg_ref, g_ref, deg_ref, b_ref, w_ref, o_ref):
    dinv, t = _relu_comb(agg_ref, g_ref, deg_ref, b_ref)
    o_ref[...] = _dot(t, w_ref[...]) * dinv


def _fused_keep_body(agg_ref, g_ref, deg_ref, b_ref, w_ref, h_ref, o_ref):
    dinv, t = _relu_comb(agg_ref, g_ref, deg_ref, b_ref)
    h_ref[...] = t
    o_ref[...] = _dot(t, w_ref[...]) * dinv


def _fused_cond_body(thresh, agg_ref, g_ref, deg_ref, b_ref, hp_ref, nl_ref,
                     w_ref, h_ref, o_ref):
    dinv, t = _relu_comb(agg_ref, g_ref, deg_ref, b_ref)
    t = jnp.where(nl_ref[0, 0] > thresh, t, hp_ref[...])
    h_ref[...] = t
    o_ref[...] = _dot(t, w_ref[...]) * dinv


def _fused_out_body(agg_ref, g_ref, deg_ref, b_ref, hp_ref, nl_ref,
                    w_ref, bo_ref, o_ref):
    _, t = _relu_comb(agg_ref, g_ref, deg_ref, b_ref)
    t = jnp.where(nl_ref[0, 0] > 3, t, hp_ref[...])
    o_ref[...] = _dot(t, w_ref[...]) + bo_ref[...]


def _spec_base(r, d, n_pad_unused=None):
    return [
        pl.BlockSpec((NC, r, d), lambda i: (0, i, 0)),
        pl.BlockSpec((r, d), lambda i: (i, 0)),
        pl.BlockSpec((r, 1), lambda i: (i, 0)),
        pl.BlockSpec((1, d), lambda i: (0, 0)),
    ]


def _fused_mid(agg, g, deg, b, w, r):
    n, d = g.shape
    return pl.pallas_call(
        _fused_mid_body,
        grid=(n // r,),
        in_specs=_spec_base(r, d) + [pl.BlockSpec((d, d), lambda i: (0, 0))],
        out_specs=pl.BlockSpec((r, d), lambda i: (i, 0)),
        out_shape=jax.ShapeDtypeStruct((n, d), jnp.float32),
    )(agg, g, deg, b.reshape(1, d), w)


def _fused_keep(agg, g, deg, b, w, r):
    n, d = g.shape
    return pl.pallas_call(
        _fused_keep_body,
        grid=(n // r,),
        in_specs=_spec_base(r, d) + [pl.BlockSpec((d, d), lambda i: (0, 0))],
        out_specs=[pl.BlockSpec((r, d), lambda i: (i, 0)),
                   pl.BlockSpec((r, d), lambda i: (i, 0))],
        out_shape=[jax.ShapeDtypeStruct((n, d), jnp.float32),
                   jax.ShapeDtypeStruct((n, d), jnp.float32)],
    )(agg, g, deg, b.reshape(1, d), w)


def _fused_cond(agg, g, deg, b, hp, nl, w, r, thresh):
    n, d = g.shape
    return pl.pallas_call(
        functools.partial(_fused_cond_body, thresh),
        grid=(n // r,),
        in_specs=_spec_base(r, d) + [
            pl.BlockSpec((r, d), lambda i: (i, 0)),
            pl.BlockSpec((1, 1), lambda i: (0, 0)),
            pl.BlockSpec((d, d), lambda i: (0, 0)),
        ],
        out_specs=[pl.BlockSpec((r, d), lambda i: (i, 0)),
                   pl.BlockSpec((r, d), lambda i: (i, 0))],
        out_shape=[jax.ShapeDtypeStruct((n, d), jnp.float32),
                   jax.ShapeDtypeStruct((n, d), jnp.float32)],
    )(agg, g, deg, b.reshape(1, d), hp, nl, w)


def _fused_out(agg, g, deg, b, hp, nl, w, bo, r):
    n, d = g.shape
    return pl.pallas_call(
        _fused_out_body,
        grid=(n // r,),
        in_specs=_spec_base(r, d) + [
            pl.BlockSpec((r, d), lambda i: (i, 0)),
            pl.BlockSpec((1, 1), lambda i: (0, 0)),
            pl.BlockSpec((d, 1), lambda i: (0, 0)),
            pl.BlockSpec((1, 1), lambda i: (0, 0)),
        ],
        out_specs=pl.BlockSpec((r, 1), lambda i: (i, 0)),
        out_shape=jax.ShapeDtypeStruct((n, 1), jnp.float32),
    )(agg, g, deg, b.reshape(1, d), hp, nl, w, bo.reshape(1, 1))


def kernel(num_layers, x, edge_index, W0, b0, W1, b1, W2, b2, W3, b3, W4, b4,
           Wout, bout):
    n, d = x.shape
    e = edge_index.shape[1]
    n_pad = _ceil_to(n + 1, NS * CH)
    epw = _ceil_to(-(-e // NW), SB * CH)
    k = epw // CH
    e_pad = epw * NW
    r = 1000

    src = edge_index[0].astype(jnp.int32)
    dst = edge_index[1].astype(jnp.int32)
    npad_e = e_pad - e
    ar = jnp.arange(npad_e, dtype=jnp.int32)
    src_p = jnp.concatenate([src, ar % n]).reshape(NW, k, CH)
    dst_p = jnp.concatenate([dst, n + ar % (n_pad - n)]).reshape(NW, k, CH)

    zerosd = jnp.zeros((CH, d), jnp.float32)
    onesd = jnp.ones((CH, d), jnp.float32)
    nl = jnp.full((1, 1), num_layers, jnp.int32)

    degp = _make_deg_kernel(n_pad, k, d)(dst_p, onesd, zerosd)
    u0 = _mm(x, W0, r)
    g0, deg = _scale0(u0, degp, r)

    agg = _make_agg_kernel(n, n_pad, k, d)

    def do_agg(g):
        return agg(g, src_p, dst_p, zerosd)

    a0 = do_agg(g0)
    g1 = _fused_mid(a0, g0, deg, b0, W1, r)
    a1 = do_agg(g1)
    h2, g2 = _fused_keep(a1, g1, deg, b1, W2, r)
    a2 = do_agg(g2)
    h3, g3 = _fused_cond(a2, g2, deg, b2, h2, nl, W3, r, 1)
    a3 = do_agg(g3)
    h4, g4 = _fused_cond(a3, g3, deg, b3, h3, nl, W4, r, 2)
    a4 = do_agg(g4)
    return _fused_out(a4, g4, deg, b4, h4, nl, Wout, bout, r)


# speedup vs baseline: 20.8215x; 1.1546x over previous
import functools

import jax
import jax.numpy as jnp
from jax import lax
from jax.experimental import pallas as pl
from jax.experimental.pallas import tpu as pltpu
from jax.experimental.pallas import tpu_sc as plsc

NC = 2
NS = 16
NW = NC * NS
CH = 128
CHE = 64
SB = 16

_MESH = plsc.VectorSubcoreMesh(core_axis_name="c", subcore_axis_name="s")


def _ceil_to(v, m):
    return -(-v // m) * m


@functools.lru_cache(maxsize=None)
def _make_agg_kernel(n, n_pad, k, d):
    rps = n_pad // NS
    zb = rps // CH

    @functools.partial(
        pl.kernel,
        mesh=_MESH,
        out_type=jax.ShapeDtypeStruct((NC, n_pad, d), jnp.float32),
        scratch_types=[
            pltpu.VMEM((SB, CHE), jnp.int32),
            pltpu.VMEM((SB, CHE), jnp.int32),
            pltpu.VMEM((CHE, d), jnp.float32),
            pltpu.VMEM((CHE, d), jnp.float32),
            pltpu.VMEM((CHE, d), jnp.float32),
            pltpu.VMEM((CHE, d), jnp.float32),
            pltpu.VMEM_SHARED((n_pad, d), jnp.float32),
            pltpu.SemaphoreType.DMA,
            pltpu.SemaphoreType.DMA,
        ],
    )
    def agg_kernel(g_hbm, src_hbm, dst_hbm, zeros_hbm, out_hbm,
                   src_v, dst_v, rows_a, rows_b, rows_c, rows_d, shared, gsem, ssem):
        c = lax.axis_index("c")
        s = lax.axis_index("s")
        wid = c * NS + s

        @pl.loop(0, zb)
        def _(z):
            pltpu.sync_copy(zeros_hbm, shared.at[pl.ds(s * rps + z * CH, CH)])

        plsc.subcore_barrier()

        @pl.loop(0, k // SB)
        def _(jb):
            pltpu.sync_copy(src_hbm.at[wid].at[pl.ds(jb * SB, SB)], src_v)
            pltpu.sync_copy(dst_hbm.at[wid].at[pl.ds(jb * SB, SB)], dst_v)
            bufs = (rows_a, rows_b, rows_c, rows_d)
            gh = {j: pltpu.async_copy(g_hbm.at[src_v.at[j]], bufs[j], gsem)
                  for j in range(2)}
            sh = {}
            for j in range(SB):
                if j >= 2:
                    sh[j - 2].wait()
                if j + 2 < SB:
                    gh[j + 2] = pltpu.async_copy(
                        g_hbm.at[src_v.at[j + 2]], bufs[(j + 2) % 4], gsem)
                gh[j].wait()
                sh[j] = pltpu.async_copy(
                    bufs[j % 4], shared.at[dst_v.at[j]], ssem, add=True)
            sh[SB - 2].wait()
            sh[SB - 1].wait()

        plsc.subcore_barrier()

        @pl.loop(0, zb)
        def _(z):
            r0 = s * rps + z * CH
            pltpu.sync_copy(shared.at[pl.ds(r0, CH)], out_hbm.at[c].at[pl.ds(r0, CH)])

    return agg_kernel


@functools.lru_cache(maxsize=None)
def _make_deg_kernel(n_pad, k, d):
    rps = n_pad // NS
    zb = rps // CH

    @functools.partial(
        pl.kernel,
        mesh=_MESH,
        out_type=jax.ShapeDtypeStruct((NC, n_pad, d), jnp.float32),
        scratch_types=[
            pltpu.VMEM((SB, CHE), jnp.int32),
            pltpu.VMEM((CHE, d), jnp.float32),
            pltpu.VMEM_SHARED((n_pad, d), jnp.float32),
            pltpu.SemaphoreType.DMA,
        ],
    )
    def deg_kernel(dst_hbm, ones_hbm, zeros_hbm, out_hbm,
                   dst_v, ones_v, shared, ssem):
        c = lax.axis_index("c")
        s = lax.axis_index("s")
        wid = c * NS + s
        pltpu.sync_copy(ones_hbm, ones_v)

        @pl.loop(0, zb)
        def _(z):
            pltpu.sync_copy(zeros_hbm, shared.at[pl.ds(s * rps + z * CH, CH)])

        plsc.subcore_barrier()

        @pl.loop(0, k // SB)
        def _(jb):
            pltpu.sync_copy(dst_hbm.at[wid].at[pl.ds(jb * SB, SB)], dst_v)
            sh = {}
            for j in range(SB):
                sh[j] = pltpu.async_copy(
                    ones_v, shared.at[dst_v.at[j]], ssem, add=True)
                if j >= 1:
                    sh[j - 1].wait()
            sh[SB - 1].wait()

        plsc.subcore_barrier()

        @pl.loop(0, zb)
        def _(z):
            r0 = s * rps + z * CH
            pltpu.sync_copy(shared.at[pl.ds(r0, CH)], out_hbm.at[c].at[pl.ds(r0, CH)])

    return deg_kernel


def _dot(a, b):
    return jnp.dot(a, b, preferred_element_type=jnp.float32)


def _mm_body(a_ref, w_ref, o_ref):
    o_ref[...] = _dot(a_ref[...], w_ref[...])


def _mm(a, w, r):
    n, d = a.shape
    return pl.pallas_call(
        _mm_body,
        grid=(n // r,),
        in_specs=[
            pl.BlockSpec((r, d), lambda i: (i, 0)),
            pl.BlockSpec((d, d), lambda i: (0, 0)),
        ],
        out_specs=pl.BlockSpec((r, d), lambda i: (i, 0)),
        out_shape=jax.ShapeDtypeStruct((n, d), jnp.float32),
    )(a, w)


def _scale0_body(u_ref, degp_ref, o_ref, deg_ref):
    deg = degp_ref[0, :, 0:1] + degp_ref[1, :, 0:1] + 1.0
    deg_ref[...] = deg
    o_ref[...] = u_ref[...] * lax.rsqrt(deg)


def _scale0(u, degp, r):
    n, d = u.shape
    return pl.pallas_call(
        _scale0_body,
        grid=(n // r,),
        in_specs=[
            pl.BlockSpec((r, d), lambda i: (i, 0)),
            pl.BlockSpec((NC, r, d), lambda i: (0, i, 0)),
        ],
        out_specs=[pl.BlockSpec((r, d), lambda i: (i, 0)),
                   pl.BlockSpec((r, 1), lambda i: (i, 0))],
        out_shape=[jax.ShapeDtypeStruct((n, d), jnp.float32),
                   jax.ShapeDtypeStruct((n, 1), jnp.float32)],
    )(u, degp)


def _relu_comb(agg_ref, g_ref, deg_ref, b_ref):
    dinv = lax.rsqrt(deg_ref[...])
    return dinv, jnp.maximum(
        (agg_ref[0] + agg_ref[1] + g_ref[...]) * dinv + b_ref[...], 0.0)


def _fused_mid_body(agg_ref, g_ref, deg_ref, b_ref, w_ref, o_ref):
    dinv, t = _relu_comb(agg_ref, g_ref, deg_ref, b_ref)
    o_ref[...] = _dot(t, w_ref[...]) * dinv


def _fused_keep_body(agg_ref, g_ref, deg_ref, b_ref, w_ref, h_ref, o_ref):
    dinv, t = _relu_comb(agg_ref, g_ref, deg_ref, b_ref)
    h_ref[...] = t
    o_ref[...] = _dot(t, w_ref[...]) * dinv


def _fused_cond_body(thresh, agg_ref, g_ref, deg_ref, b_ref, hp_ref, nl_ref,
                     w_ref, h_ref, o_ref):
    dinv, t = _relu_comb(agg_ref, g_ref, deg_ref, b_ref)
    t = jnp.where(nl_ref[0, 0] > thresh, t, hp_ref[...])
    h_ref[...] = t
    o_ref[...] = _dot(t, w_ref[...]) * dinv


def _fused_out_body(agg_ref, g_ref, deg_ref, b_ref, hp_ref, nl_ref,
                    w_ref, bo_ref, o_ref):
    _, t = _relu_comb(agg_ref, g_ref, deg_ref, b_ref)
    t = jnp.where(nl_ref[0, 0] > 3, t, hp_ref[...])
    o_ref[...] = _dot(t, w_ref[...]) + bo_ref[...]


def _spec_base(r, d, n_pad_unused=None):
    return [
        pl.BlockSpec((NC, r, d), lambda i: (0, i, 0)),
        pl.BlockSpec((r, d), lambda i: (i, 0)),
        pl.BlockSpec((r, 1), lambda i: (i, 0)),
        pl.BlockSpec((1, d), lambda i: (0, 0)),
    ]


def _fused_mid(agg, g, deg, b, w, r):
    n, d = g.shape
    return pl.pallas_call(
        _fused_mid_body,
        grid=(n // r,),
        in_specs=_spec_base(r, d) + [pl.BlockSpec((d, d), lambda i: (0, 0))],
        out_specs=pl.BlockSpec((r, d), lambda i: (i, 0)),
        out_shape=jax.ShapeDtypeStruct((n, d), jnp.float32),
    )(agg, g, deg, b.reshape(1, d), w)


def _fused_keep(agg, g, deg, b, w, r):
    n, d = g.shape
    return pl.pallas_call(
        _fused_keep_body,
        grid=(n // r,),
        in_specs=_spec_base(r, d) + [pl.BlockSpec((d, d), lambda i: (0, 0))],
        out_specs=[pl.BlockSpec((r, d), lambda i: (i, 0)),
                   pl.BlockSpec((r, d), lambda i: (i, 0))],
        out_shape=[jax.ShapeDtypeStruct((n, d), jnp.float32),
                   jax.ShapeDtypeStruct((n, d), jnp.float32)],
    )(agg, g, deg, b.reshape(1, d), w)


def _fused_cond(agg, g, deg, b, hp, nl, w, r, thresh):
    n, d = g.shape
    return pl.pallas_call(
        functools.partial(_fused_cond_body, thresh),
        grid=(n // r,),
        in_specs=_spec_base(r, d) + [
            pl.BlockSpec((r, d), lambda i: (i, 0)),
            pl.BlockSpec((1, 1), lambda i: (0, 0)),
            pl.BlockSpec((d, d), lambda i: (0, 0)),
        ],
        out_specs=[pl.BlockSpec((r, d), lambda i: (i, 0)),
                   pl.BlockSpec((r, d), lambda i: (i, 0))],
        out_shape=[jax.ShapeDtypeStruct((n, d), jnp.float32),
                   jax.ShapeDtypeStruct((n, d), jnp.float32)],
    )(agg, g, deg, b.reshape(1, d), hp, nl, w)


def _fused_out(agg, g, deg, b, hp, nl, w, bo, r):
    n, d = g.shape
    return pl.pallas_call(
        _fused_out_body,
        grid=(n // r,),
        in_specs=_spec_base(r, d) + [
            pl.BlockSpec((r, d), lambda i: (i, 0)),
            pl.BlockSpec((1, 1), lambda i: (0, 0)),
            pl.BlockSpec((d, 1), lambda i: (0, 0)),
            pl.BlockSpec((1, 1), lambda i: (0, 0)),
        ],
        out_specs=pl.BlockSpec((r, 1), lambda i: (i, 0)),
        out_shape=jax.ShapeDtypeStruct((n, 1), jnp.float32),
    )(agg, g, deg, b.reshape(1, d), hp, nl, w, bo.reshape(1, 1))


def kernel(num_layers, x, edge_index, W0, b0, W1, b1, W2, b2, W3, b3, W4, b4,
           Wout, bout):
    n, d = x.shape
    e = edge_index.shape[1]
    n_pad = _ceil_to(n + 1, NS * CH)
    epw = _ceil_to(-(-e // NW), SB * CHE)
    k = epw // CHE
    e_pad = epw * NW
    r = 1000

    src = edge_index[0].astype(jnp.int32)
    dst = edge_index[1].astype(jnp.int32)
    npad_e = e_pad - e
    ar = jnp.arange(npad_e, dtype=jnp.int32)
    src_p = jnp.concatenate([src, ar % n]).reshape(NW, k, CHE)
    dst_p = jnp.concatenate([dst, n + ar % (n_pad - n)]).reshape(NW, k, CHE)

    zerosd = jnp.zeros((CH, d), jnp.float32)
    onesd = jnp.ones((CHE, d), jnp.float32)
    nl = jnp.full((1, 1), num_layers, jnp.int32)

    degp = _make_deg_kernel(n_pad, k, d)(dst_p, onesd, zerosd)
    u0 = _mm(x, W0, r)
    g0, deg = _scale0(u0, degp, r)

    agg = _make_agg_kernel(n, n_pad, k, d)

    def do_agg(g):
        return agg(g, src_p, dst_p, zerosd)

    a0 = do_agg(g0)
    g1 = _fused_mid(a0, g0, deg, b0, W1, r)
    a1 = do_agg(g1)
    h2, g2 = _fused_keep(a1, g1, deg, b1, W2, r)
    a2 = do_agg(g2)
    h3, g3 = _fused_cond(a2, g2, deg, b2, h2, nl, W3, r, 1)
    a3 = do_agg(g3)
    h4, g4 = _fused_cond(a3, g3, deg, b3, h3, nl, W4, r, 2)
    a4 = do_agg(g4)
    return _fused_out(a4, g4, deg, b4, h4, nl, Wout, bout, r)


# speedup vs baseline: 21.8303x; 1.0485x over previous
import functools

import jax
import jax.numpy as jnp
from jax import lax
from jax.experimental import pallas as pl
from jax.experimental.pallas import tpu as pltpu
from jax.experimental.pallas import tpu_sc as plsc

NC = 2
NS = 16
NW = NC * NS
CH = 128
CHE = 64
SB = 32

_MESH = plsc.VectorSubcoreMesh(core_axis_name="c", subcore_axis_name="s")


def _ceil_to(v, m):
    return -(-v // m) * m


@functools.lru_cache(maxsize=None)
def _make_agg_kernel(n, n_pad, k, d):
    rps = n_pad // NS
    zb = rps // CH

    @functools.partial(
        pl.kernel,
        mesh=_MESH,
        out_type=jax.ShapeDtypeStruct((NC, n_pad, d), jnp.float32),
        scratch_types=[
            pltpu.VMEM((SB, CHE), jnp.int32),
            pltpu.VMEM((SB, CHE), jnp.int32),
            pltpu.VMEM((CHE, d), jnp.float32),
            pltpu.VMEM((CHE, d), jnp.float32),
            pltpu.VMEM((CHE, d), jnp.float32),
            pltpu.VMEM((CHE, d), jnp.float32),
            pltpu.VMEM((CHE, d), jnp.float32),
            pltpu.VMEM_SHARED((n_pad, d), jnp.float32),
            pltpu.SemaphoreType.DMA,
            pltpu.SemaphoreType.DMA,
        ],
    )
    def agg_kernel(g_hbm, src_hbm, dst_hbm, zeros_hbm, out_hbm,
                   src_v, dst_v, rows_a, rows_b, rows_c, rows_d, rows_e,
                   shared, gsem, ssem):
        c = lax.axis_index("c")
        s = lax.axis_index("s")
        wid = c * NS + s

        @pl.loop(0, zb)
        def _(z):
            pltpu.sync_copy(zeros_hbm, shared.at[pl.ds(s * rps + z * CH, CH)])

        plsc.subcore_barrier()

        @pl.loop(0, k // SB)
        def _(jb):
            pltpu.sync_copy(src_hbm.at[wid].at[pl.ds(jb * SB, SB)], src_v)
            pltpu.sync_copy(dst_hbm.at[wid].at[pl.ds(jb * SB, SB)], dst_v)
            bufs = (rows_a, rows_b, rows_c, rows_d, rows_e)
            gh = {j: pltpu.async_copy(g_hbm.at[src_v.at[j]], bufs[j], gsem)
                  for j in range(2)}
            sh = {}
            for j in range(SB):
                if j >= 3:
                    sh[j - 3].wait()
                if j + 2 < SB:
                    gh[j + 2] = pltpu.async_copy(
                        g_hbm.at[src_v.at[j + 2]], bufs[(j + 2) % 5], gsem)
                gh[j].wait()
                sh[j] = pltpu.async_copy(
                    bufs[j % 5], shared.at[dst_v.at[j]], ssem, add=True)
            for j in range(max(0, SB - 3), SB):
                sh[j].wait()

        plsc.subcore_barrier()

        @pl.loop(0, zb)
        def _(z):
            r0 = s * rps + z * CH
            pltpu.sync_copy(shared.at[pl.ds(r0, CH)], out_hbm.at[c].at[pl.ds(r0, CH)])

    return agg_kernel


@functools.lru_cache(maxsize=None)
def _make_deg_kernel(n_pad, k, d):
    rps = n_pad // NS
    zb = rps // CH

    @functools.partial(
        pl.kernel,
        mesh=_MESH,
        out_type=jax.ShapeDtypeStruct((NC, n_pad, d), jnp.float32),
        scratch_types=[
            pltpu.VMEM((SB, CHE), jnp.int32),
            pltpu.VMEM((CHE, d), jnp.float32),
            pltpu.VMEM_SHARED((n_pad, d), jnp.float32),
            pltpu.SemaphoreType.DMA,
        ],
    )
    def deg_kernel(dst_hbm, ones_hbm, zeros_hbm, out_hbm,
                   dst_v, ones_v, shared, ssem):
        c = lax.axis_index("c")
        s = lax.axis_index("s")
        wid = c * NS + s
        pltpu.sync_copy(ones_hbm, ones_v)

        @pl.loop(0, zb)
        def _(z):
            pltpu.sync_copy(zeros_hbm, shared.at[pl.ds(s * rps + z * CH, CH)])

        plsc.subcore_barrier()

        @pl.loop(0, k // SB)
        def _(jb):
            pltpu.sync_copy(dst_hbm.at[wid].at[pl.ds(jb * SB, SB)], dst_v)
            sh = {}
            for j in range(SB):
                sh[j] = pltpu.async_copy(
                    ones_v, shared.at[dst_v.at[j]], ssem, add=True)
                if j >= 4:
                    sh[j - 4].wait()
            for j in range(max(0, SB - 4), SB):
                sh[j].wait()

        plsc.subcore_barrier()

        @pl.loop(0, zb)
        def _(z):
            r0 = s * rps + z * CH
            pltpu.sync_copy(shared.at[pl.ds(r0, CH)], out_hbm.at[c].at[pl.ds(r0, CH)])

    return deg_kernel


def _dot(a, b):
    return jnp.dot(a, b, preferred_element_type=jnp.float32)


def _mm_body(a_ref, w_ref, o_ref):
    o_ref[...] = _dot(a_ref[...], w_ref[...])


def _mm(a, w, r):
    n, d = a.shape
    return pl.pallas_call(
        _mm_body,
        grid=(n // r,),
        in_specs=[
            pl.BlockSpec((r, d), lambda i: (i, 0)),
            pl.BlockSpec((d, d), lambda i: (0, 0)),
        ],
        out_specs=pl.BlockSpec((r, d), lambda i: (i, 0)),
        out_shape=jax.ShapeDtypeStruct((n, d), jnp.float32),
    )(a, w)


def _scale0_body(u_ref, degp_ref, o_ref, deg_ref):
    deg = degp_ref[0, :, 0:1] + degp_ref[1, :, 0:1] + 1.0
    deg_ref[...] = deg
    o_ref[...] = u_ref[...] * lax.rsqrt(deg)


def _scale0(u, degp, r):
    n, d = u.shape
    return pl.pallas_call(
        _scale0_body,
        grid=(n // r,),
        in_specs=[
            pl.BlockSpec((r, d), lambda i: (i, 0)),
            pl.BlockSpec((NC, r, d), lambda i: (0, i, 0)),
        ],
        out_specs=[pl.BlockSpec((r, d), lambda i: (i, 0)),
                   pl.BlockSpec((r, 1), lambda i: (i, 0))],
        out_shape=[jax.ShapeDtypeStruct((n, d), jnp.float32),
                   jax.ShapeDtypeStruct((n, 1), jnp.float32)],
    )(u, degp)


def _relu_comb(agg_ref, g_ref, deg_ref, b_ref):
    dinv = lax.rsqrt(deg_ref[...])
    return dinv, jnp.maximum(
        (agg_ref[0] + agg_ref[1] + g_ref[...]) * dinv + b_ref[...], 0.0)


def _fused_mid_body(agg_ref, g_ref, deg_ref, b_ref, w_ref, o_ref):
    dinv, t = _relu_comb(agg_ref, g_ref, deg_ref, b_ref)
    o_ref[...] = _dot(t, w_ref[...]) * dinv


def _fused_keep_body(agg_ref, g_ref, deg_ref, b_ref, w_ref, h_ref, o_ref):
    dinv, t = _relu_comb(agg_ref, g_ref, deg_ref, b_ref)
    h_ref[...] = t
    o_ref[...] = _dot(t, w_ref[...]) * dinv


def _fused_cond_body(thresh, agg_ref, g_ref, deg_ref, b_ref, hp_ref, nl_ref,
                     w_ref, h_ref, o_ref):
    dinv, t = _relu_comb(agg_ref, g_ref, deg_ref, b_ref)
    t = jnp.where(nl_ref[0, 0] > thresh, t, hp_ref[...])
    h_ref[...] = t
    o_ref[...] = _dot(t, w_ref[...]) * dinv


def _fused_out_body(agg_ref, g_ref, deg_ref, b_ref, hp_ref, nl_ref,
                    w_ref, bo_ref, o_ref):
    _, t = _relu_comb(agg_ref, g_ref, deg_ref, b_ref)
    t = jnp.where(nl_ref[0, 0] > 3, t, hp_ref[...])
    o_ref[...] = _dot(t, w_ref[...]) + bo_ref[...]


def _spec_base(r, d, n_pad_unused=None):
    return [
        pl.BlockSpec((NC, r, d), lambda i: (0, i, 0)),
        pl.BlockSpec((r, d), lambda i: (i, 0)),
        pl.BlockSpec((r, 1), lambda i: (i, 0)),
        pl.BlockSpec((1, d), lambda i: (0, 0)),
    ]


def _fused_mid(agg, g, deg, b, w, r):
    n, d = g.shape
    return pl.pallas_call(
        _fused_mid_body,
        grid=(n // r,),
        in_specs=_spec_base(r, d) + [pl.BlockSpec((d, d), lambda i: (0, 0))],
        out_specs=pl.BlockSpec((r, d), lambda i: (i, 0)),
        out_shape=jax.ShapeDtypeStruct((n, d), jnp.float32),
    )(agg, g, deg, b.reshape(1, d), w)


def _fused_keep(agg, g, deg, b, w, r):
    n, d = g.shape
    return pl.pallas_call(
        _fused_keep_body,
        grid=(n // r,),
        in_specs=_spec_base(r, d) + [pl.BlockSpec((d, d), lambda i: (0, 0))],
        out_specs=[pl.BlockSpec((r, d), lambda i: (i, 0)),
                   pl.BlockSpec((r, d), lambda i: (i, 0))],
        out_shape=[jax.ShapeDtypeStruct((n, d), jnp.float32),
                   jax.ShapeDtypeStruct((n, d), jnp.float32)],
    )(agg, g, deg, b.reshape(1, d), w)


def _fused_cond(agg, g, deg, b, hp, nl, w, r, thresh):
    n, d = g.shape
    return pl.pallas_call(
        functools.partial(_fused_cond_body, thresh),
        grid=(n // r,),
        in_specs=_spec_base(r, d) + [
            pl.BlockSpec((r, d), lambda i: (i, 0)),
            pl.BlockSpec((1, 1), lambda i: (0, 0)),
            pl.BlockSpec((d, d), lambda i: (0, 0)),
        ],
        out_specs=[pl.BlockSpec((r, d), lambda i: (i, 0)),
                   pl.BlockSpec((r, d), lambda i: (i, 0))],
        out_shape=[jax.ShapeDtypeStruct((n, d), jnp.float32),
                   jax.ShapeDtypeStruct((n, d), jnp.float32)],
    )(agg, g, deg, b.reshape(1, d), hp, nl, w)


def _fused_out(agg, g, deg, b, hp, nl, w, bo, r):
    n, d = g.shape
    return pl.pallas_call(
        _fused_out_body,
        grid=(n // r,),
        in_specs=_spec_base(r, d) + [
            pl.BlockSpec((r, d), lambda i: (i, 0)),
            pl.BlockSpec((1, 1), lambda i: (0, 0)),
            pl.BlockSpec((d, 1), lambda i: (0, 0)),
            pl.BlockSpec((1, 1), lambda i: (0, 0)),
        ],
        out_specs=pl.BlockSpec((r, 1), lambda i: (i, 0)),
        out_shape=jax.ShapeDtypeStruct((n, 1), jnp.float32),
    )(agg, g, deg, b.reshape(1, d), hp, nl, w, bo.reshape(1, 1))


def kernel(num_layers, x, edge_index, W0, b0, W1, b1, W2, b2, W3, b3, W4, b4,
           Wout, bout):
    n, d = x.shape
    e = edge_index.shape[1]
    n_pad = _ceil_to(n + 1, NS * CH)
    epw = _ceil_to(-(-e // NW), SB * CHE)
    k = epw // CHE
    e_pad = epw * NW
    r = 1000

    src = edge_index[0].astype(jnp.int32)
    dst = edge_index[1].astype(jnp.int32)
    npad_e = e_pad - e
    ar = jnp.arange(npad_e, dtype=jnp.int32)
    src_p = jnp.concatenate([src, ar % n]).reshape(NW, k, CHE)
    dst_p = jnp.concatenate([dst, n + ar % (n_pad - n)]).reshape(NW, k, CHE)

    zerosd = jnp.zeros((CH, d), jnp.float32)
    onesd = jnp.ones((CHE, d), jnp.float32)
    nl = jnp.full((1, 1), num_layers, jnp.int32)

    degp = _make_deg_kernel(n_pad, k, d)(dst_p, onesd, zerosd)
    u0 = _mm(x, W0, r)
    g0, deg = _scale0(u0, degp, r)

    agg = _make_agg_kernel(n, n_pad, k, d)

    def do_agg(g):
        return agg(g, src_p, dst_p, zerosd)

    a0 = do_agg(g0)
    g1 = _fused_mid(a0, g0, deg, b0, W1, r)
    a1 = do_agg(g1)
    h2, g2 = _fused_keep(a1, g1, deg, b1, W2, r)
    a2 = do_agg(g2)
    h3, g3 = _fused_cond(a2, g2, deg, b2, h2, nl, W3, r, 1)
    a3 = do_agg(g3)
    h4, g4 = _fused_cond(a3, g3, deg, b3, h3, nl, W4, r, 2)
    a4 = do_agg(g4)
    return _fused_out(a4, g4, deg, b4, h4, nl, Wout, bout, r)


# speedup vs baseline: 21.9885x; 1.0072x over previous
import functools

import jax
import jax.numpy as jnp
from jax import lax
from jax.experimental import pallas as pl
from jax.experimental.pallas import tpu as pltpu
from jax.experimental.pallas import tpu_sc as plsc

NC = 2
NS = 16
NW = NC * NS
CH = 128
CHE = 64
SB = 32

_MESH = plsc.VectorSubcoreMesh(core_axis_name="c", subcore_axis_name="s")


def _ceil_to(v, m):
    return -(-v // m) * m


@functools.lru_cache(maxsize=None)
def _make_agg_kernel(n, n_pad, k, d):
    rps = n_pad // NS
    zb = rps // CH

    @functools.partial(
        pl.kernel,
        mesh=_MESH,
        out_type=jax.ShapeDtypeStruct((NC, n_pad, d), jnp.float32),
        scratch_types=[
            pltpu.VMEM((SB, CHE), jnp.int32),
            pltpu.VMEM((SB, CHE), jnp.int32),
            pltpu.VMEM((CHE, d), jnp.float32),
            pltpu.VMEM((CHE, d), jnp.float32),
            pltpu.VMEM((CHE, d), jnp.float32),
            pltpu.VMEM((CHE, d), jnp.float32),
            pltpu.VMEM((CHE, d), jnp.float32),
            pltpu.VMEM_SHARED((n_pad, d), jnp.float32),
            pltpu.SemaphoreType.DMA,
            pltpu.SemaphoreType.DMA,
        ],
    )
    def agg_kernel(g_hbm, src_hbm, dst_hbm, zeros_hbm, out_hbm,
                   src_v, dst_v, rows_a, rows_b, rows_c, rows_d, rows_e,
                   shared, gsem, ssem):
        c = lax.axis_index("c")
        s = lax.axis_index("s")
        wid = c * NS + s

        @pl.loop(0, zb)
        def _(z):
            pltpu.sync_copy(zeros_hbm, shared.at[pl.ds(s * rps + z * CH, CH)])

        plsc.subcore_barrier()

        @pl.loop(0, k // SB)
        def _(jb):
            pltpu.sync_copy(src_hbm.at[wid].at[pl.ds(jb * SB, SB)], src_v)
            pltpu.sync_copy(dst_hbm.at[wid].at[pl.ds(jb * SB, SB)], dst_v)
            bufs = (rows_a, rows_b, rows_c, rows_d, rows_e)
            gh = {j: pltpu.async_copy(g_hbm.at[src_v.at[j]], bufs[j], gsem)
                  for j in range(3)}
            sh = {}
            for j in range(SB):
                if j >= 2:
                    sh[j - 2].wait()
                if j + 3 < SB:
                    gh[j + 3] = pltpu.async_copy(
                        g_hbm.at[src_v.at[j + 3]], bufs[(j + 3) % 5], gsem)
                gh[j].wait()
                sh[j] = pltpu.async_copy(
                    bufs[j % 5], shared.at[dst_v.at[j]], ssem, add=True)
            for j in range(max(0, SB - 2), SB):
                sh[j].wait()

        plsc.subcore_barrier()

        @pl.loop(0, zb)
        def _(z):
            r0 = s * rps + z * CH
            pltpu.sync_copy(shared.at[pl.ds(r0, CH)], out_hbm.at[c].at[pl.ds(r0, CH)])

    return agg_kernel


@functools.lru_cache(maxsize=None)
def _make_deg_kernel(n_pad, k, d):
    rps = n_pad // NS
    zb = rps // CH

    @functools.partial(
        pl.kernel,
        mesh=_MESH,
        out_type=jax.ShapeDtypeStruct((NC, n_pad, d), jnp.float32),
        scratch_types=[
            pltpu.VMEM((SB, CHE), jnp.int32),
            pltpu.VMEM((CHE, d), jnp.float32),
            pltpu.VMEM_SHARED((n_pad, d), jnp.float32),
            pltpu.SemaphoreType.DMA,
        ],
    )
    def deg_kernel(dst_hbm, ones_hbm, zeros_hbm, out_hbm,
                   dst_v, ones_v, shared, ssem):
        c = lax.axis_index("c")
        s = lax.axis_index("s")
        wid = c * NS + s
        pltpu.sync_copy(ones_hbm, ones_v)

        @pl.loop(0, zb)
        def _(z):
            pltpu.sync_copy(zeros_hbm, shared.at[pl.ds(s * rps + z * CH, CH)])

        plsc.subcore_barrier()

        @pl.loop(0, k // SB)
        def _(jb):
            pltpu.sync_copy(dst_hbm.at[wid].at[pl.ds(jb * SB, SB)], dst_v)
            sh = {}
            for j in range(SB):
                sh[j] = pltpu.async_copy(
                    ones_v, shared.at[dst_v.at[j]], ssem, add=True)
                if j >= 4:
                    sh[j - 4].wait()
            for j in range(max(0, SB - 4), SB):
                sh[j].wait()

        plsc.subcore_barrier()

        @pl.loop(0, zb)
        def _(z):
            r0 = s * rps + z * CH
            pltpu.sync_copy(shared.at[pl.ds(r0, CH)], out_hbm.at[c].at[pl.ds(r0, CH)])

    return deg_kernel


def _dot(a, b):
    return jnp.dot(a, b, preferred_element_type=jnp.float32)


def _mm_body(a_ref, w_ref, o_ref):
    o_ref[...] = _dot(a_ref[...], w_ref[...])


def _mm(a, w, r):
    n, d = a.shape
    return pl.pallas_call(
        _mm_body,
        grid=(n // r,),
        in_specs=[
            pl.BlockSpec((r, d), lambda i: (i, 0)),
            pl.BlockSpec((d, d), lambda i: (0, 0)),
        ],
        out_specs=pl.BlockSpec((r, d), lambda i: (i, 0)),
        out_shape=jax.ShapeDtypeStruct((n, d), jnp.float32),
    )(a, w)


def _scale0_body(u_ref, degp_ref, o_ref, deg_ref):
    deg = degp_ref[0, :, 0:1] + degp_ref[1, :, 0:1] + 1.0
    deg_ref[...] = deg
    o_ref[...] = u_ref[...] * lax.rsqrt(deg)


def _scale0(u, degp, r):
    n, d = u.shape
    return pl.pallas_call(
        _scale0_body,
        grid=(n // r,),
        in_specs=[
            pl.BlockSpec((r, d), lambda i: (i, 0)),
            pl.BlockSpec((NC, r, d), lambda i: (0, i, 0)),
        ],
        out_specs=[pl.BlockSpec((r, d), lambda i: (i, 0)),
                   pl.BlockSpec((r, 1), lambda i: (i, 0))],
        out_shape=[jax.ShapeDtypeStruct((n, d), jnp.float32),
                   jax.ShapeDtypeStruct((n, 1), jnp.float32)],
    )(u, degp)


def _relu_comb(agg_ref, g_ref, deg_ref, b_ref):
    dinv = lax.rsqrt(deg_ref[...])
    return dinv, jnp.maximum(
        (agg_ref[0] + agg_ref[1] + g_ref[...]) * dinv + b_ref[...], 0.0)


def _fused_mid_body(agg_ref, g_ref, deg_ref, b_ref, w_ref, o_ref):
    dinv, t = _relu_comb(agg_ref, g_ref, deg_ref, b_ref)
    o_ref[...] = _dot(t, w_ref[...]) * dinv


def _fused_keep_body(agg_ref, g_ref, deg_ref, b_ref, w_ref, h_ref, o_ref):
    dinv, t = _relu_comb(agg_ref, g_ref, deg_ref, b_ref)
    h_ref[...] = t
    o_ref[...] = _dot(t, w_ref[...]) * dinv


def _fused_cond_body(thresh, agg_ref, g_ref, deg_ref, b_ref, hp_ref, nl_ref,
                     w_ref, h_ref, o_ref):
    dinv, t = _relu_comb(agg_ref, g_ref, deg_ref, b_ref)
    t = jnp.where(nl_ref[0, 0] > thresh, t, hp_ref[...])
    h_ref[...] = t
    o_ref[...] = _dot(t, w_ref[...]) * dinv


def _fused_out_body(agg_ref, g_ref, deg_ref, b_ref, hp_ref, nl_ref,
                    w_ref, bo_ref, o_ref):
    _, t = _relu_comb(agg_ref, g_ref, deg_ref, b_ref)
    t = jnp.where(nl_ref[0, 0] > 3, t, hp_ref[...])
    o_ref[...] = _dot(t, w_ref[...]) + bo_ref[...]


def _spec_base(r, d, n_pad_unused=None):
    return [
        pl.BlockSpec((NC, r, d), lambda i: (0, i, 0)),
        pl.BlockSpec((r, d), lambda i: (i, 0)),
        pl.BlockSpec((r, 1), lambda i: (i, 0)),
        pl.BlockSpec((1, d), lambda i: (0, 0)),
    ]


def _fused_mid(agg, g, deg, b, w, r):
    n, d = g.shape
    return pl.pallas_call(
        _fused_mid_body,
        grid=(n // r,),
        in_specs=_spec_base(r, d) + [pl.BlockSpec((d, d), lambda i: (0, 0))],
        out_specs=pl.BlockSpec((r, d), lambda i: (i, 0)),
        out_shape=jax.ShapeDtypeStruct((n, d), jnp.float32),
    )(agg, g, deg, b.reshape(1, d), w)


def _fused_keep(agg, g, deg, b, w, r):
    n, d = g.shape
    return pl.pallas_call(
        _fused_keep_body,
        grid=(n // r,),
        in_specs=_spec_base(r, d) + [pl.BlockSpec((d, d), lambda i: (0, 0))],
        out_specs=[pl.BlockSpec((r, d), lambda i: (i, 0)),
                   pl.BlockSpec((r, d), lambda i: (i, 0))],
        out_shape=[jax.ShapeDtypeStruct((n, d), jnp.float32),
                   jax.ShapeDtypeStruct((n, d), jnp.float32)],
    )(agg, g, deg, b.reshape(1, d), w)


def _fused_cond(agg, g, deg, b, hp, nl, w, r, thresh):
    n, d = g.shape
    return pl.pallas_call(
        functools.partial(_fused_cond_body, thresh),
        grid=(n // r,),
        in_specs=_spec_base(r, d) + [
            pl.BlockSpec((r, d), lambda i: (i, 0)),
            pl.BlockSpec((1, 1), lambda i: (0, 0)),
            pl.BlockSpec((d, d), lambda i: (0, 0)),
        ],
        out_specs=[pl.BlockSpec((r, d), lambda i: (i, 0)),
                   pl.BlockSpec((r, d), lambda i: (i, 0))],
        out_shape=[jax.ShapeDtypeStruct((n, d), jnp.float32),
                   jax.ShapeDtypeStruct((n, d), jnp.float32)],
    )(agg, g, deg, b.reshape(1, d), hp, nl, w)


def _fused_out(agg, g, deg, b, hp, nl, w, bo, r):
    n, d = g.shape
    return pl.pallas_call(
        _fused_out_body,
        grid=(n // r,),
        in_specs=_spec_base(r, d) + [
            pl.BlockSpec((r, d), lambda i: (i, 0)),
            pl.BlockSpec((1, 1), lambda i: (0, 0)),
            pl.BlockSpec((d, 1), lambda i: (0, 0)),
            pl.BlockSpec((1, 1), lambda i: (0, 0)),
        ],
        out_specs=pl.BlockSpec((r, 1), lambda i: (i, 0)),
        out_shape=jax.ShapeDtypeStruct((n, 1), jnp.float32),
    )(agg, g, deg, b.reshape(1, d), hp, nl, w, bo.reshape(1, 1))


def kernel(num_layers, x, edge_index, W0, b0, W1, b1, W2, b2, W3, b3, W4, b4,
           Wout, bout):
    n, d = x.shape
    e = edge_index.shape[1]
    n_pad = _ceil_to(n + 1, NS * CH)
    epw = _ceil_to(-(-e // NW), SB * CHE)
    k = epw // CHE
    e_pad = epw * NW
    r = 1000

    src = edge_index[0].astype(jnp.int32)
    dst = edge_index[1].astype(jnp.int32)
    npad_e = e_pad - e
    ar = jnp.arange(npad_e, dtype=jnp.int32)
    src_p = jnp.concatenate([src, ar % n]).reshape(NW, k, CHE)
    dst_p = jnp.concatenate([dst, n + ar % (n_pad - n)]).reshape(NW, k, CHE)

    zerosd = jnp.zeros((CH, d), jnp.float32)
    onesd = jnp.ones((CHE, d), jnp.float32)
    nl = jnp.full((1, 1), num_layers, jnp.int32)

    degp = _make_deg_kernel(n_pad, k, d)(dst_p, onesd, zerosd)
    u0 = _mm(x, W0, r)
    g0, deg = _scale0(u0, degp, r)

    agg = _make_agg_kernel(n, n_pad, k, d)

    def do_agg(g):
        return agg(g, src_p, dst_p, zerosd)

    a0 = do_agg(g0)
    g1 = _fused_mid(a0, g0, deg, b0, W1, r)
    a1 = do_agg(g1)
    h2, g2 = _fused_keep(a1, g1, deg, b1, W2, r)
    a2 = do_agg(g2)
    h3, g3 = _fused_cond(a2, g2, deg, b2, h2, nl, W3, r, 1)
    a3 = do_agg(g3)
    h4, g4 = _fused_cond(a3, g3, deg, b3, h3, nl, W4, r, 2)
    a4 = do_agg(g4)
    return _fused_out(a4, g4, deg, b4, h4, nl, Wout, bout, r)
